# Initial kernel scaffold; baseline (speedup 1.0000x reference)
import jax
import jax.numpy as jnp
from jax.experimental import pallas as pl


def kernel(x, edge_index, W0, b0, W, bW, V, bV):
    raise NotImplementedError("write your pallas kernel here")


# speedup vs baseline: 2.8348x; 2.8348x over previous
import functools

import jax
import jax.numpy as jnp
from jax import lax
from jax.experimental import pallas as pl
from jax.experimental.pallas import tpu as pltpu
from jax.experimental.pallas import tpu_sc as plsc

N = 10000
E = 320000
D = 128
K = 2

SUB = 128
EP = 327680
DSUBS = 4
DNM = EP // (SUB * DSUBS)
SSUBS = 2
SNM = EP // (SUB * SSUBS)
NC = 2
NS = 16
NW = NC * NS
NP = 10240
RPS = NP // NS

_MESH = plsc.VectorSubcoreMesh(core_axis_name="c", subcore_axis_name="s")


def _deg_body(dst_hbm, zeros_hbm, ones_hbm, out_hbm, dst_v, ones_v, acc_sh):
    c = lax.axis_index("c")
    s = lax.axis_index("s")
    w = c * NS + s
    pltpu.sync_copy(zeros_hbm.at[pl.ds(s * RPS, RPS)],
                    acc_sh.at[pl.ds(s * RPS, RPS)])
    pltpu.sync_copy(ones_hbm, ones_v)
    plsc.subcore_barrier()
    mpw = DNM // NW

    def body(i, _):
        m = w * mpw + i
        pltpu.sync_copy(dst_hbm.at[m], dst_v)
        for j in range(DSUBS):
            pltpu.sync_copy(ones_v, acc_sh.at[dst_v.at[j]], add=True)
        return ()

    lax.fori_loop(0, mpw, body, ())
    plsc.subcore_barrier()
    pltpu.sync_copy(acc_sh.at[pl.ds(s * RPS, RPS)],
                    out_hbm.at[c, pl.ds(s * RPS, RPS)])


_deg_call = pl.kernel(
    _deg_body,
    out_type=jax.ShapeDtypeStruct((NC, NP, D), jnp.float32),
    mesh=_MESH,
    scratch_types=[
        pltpu.VMEM((DSUBS, SUB), jnp.int32),
        pltpu.VMEM((SUB, D), jnp.float32),
        pltpu.VMEM_SHARED((NP, D), jnp.float32),
    ],
)


def _spmm_body(per_core, table_hbm, src_hbm, dst_hbm, zeros_hbm, out_hbm,
               src_v, dst_v, rows_v, acc_sh, sem):
    c = lax.axis_index("c")
    s = lax.axis_index("s")
    pltpu.sync_copy(zeros_hbm.at[pl.ds(s * RPS, RPS)],
                    acc_sh.at[pl.ds(s * RPS, RPS)])
    plsc.subcore_barrier()
    if per_core:
        mpw = SNM // NS
        base = s * mpw
    else:
        mpw = SNM // NW
        base = (c * NS + s) * mpw

    def body(i, _):
        m = base + i
        pltpu.sync_copy(src_hbm.at[c, m], src_v)
        pltpu.sync_copy(dst_hbm.at[m], dst_v)
        descs = [
            pltpu.async_copy(table_hbm.at[src_v.at[j]], rows_v.at[j], sem)
            for j in range(SSUBS)
        ]
        for d_ in descs:
            d_.wait()
        for j in range(SSUBS):
            pltpu.sync_copy(rows_v.at[j], acc_sh.at[dst_v.at[j]], add=True)
        return ()

    lax.fori_loop(0, mpw, body, ())
    plsc.subcore_barrier()
    pltpu.sync_copy(acc_sh.at[pl.ds(s * RPS, RPS)],
                    out_hbm.at[c, pl.ds(s * RPS, RPS)])


def _make_spmm(per_core):
    return pl.kernel(
        functools.partial(_spmm_body, per_core),
        out_type=jax.ShapeDtypeStruct((NC, NP, D), jnp.float32),
        mesh=_MESH,
        scratch_types=[
            pltpu.VMEM((SSUBS, SUB), jnp.int32),
            pltpu.VMEM((SSUBS, SUB), jnp.int32),
            pltpu.VMEM((SSUBS, SUB, D), jnp.float32),
            pltpu.VMEM_SHARED((NP, D), jnp.float32),
            pltpu.SemaphoreType.DMA,
        ],
    )


_spmm_split = _make_spmm(False)
_spmm_perk = _make_spmm(True)


BLK = 1000
_GRID = N // BLK


def _prep_body(deg_ref, x_ref, v_ref, bv_ref, norm_ref, xn_ref, xv_ref):
    deg = deg_ref[0, :, 0:1] + deg_ref[1, :, 0:1]
    norm = lax.rsqrt(jnp.maximum(deg, 1.0))
    x = x_ref[...]
    norm_ref[...] = norm
    xn_ref[...] = x * norm
    for k in range(K):
        xv_ref[k] = (jnp.dot(x, v_ref[k], preferred_element_type=jnp.float32)
                     + bv_ref[k:k + 1, :])


_prep_call = pl.pallas_call(
    _prep_body,
    grid=(_GRID,),
    in_specs=[
        pl.BlockSpec((NC, BLK, D), lambda i: (0, i, 0)),
        pl.BlockSpec((BLK, D), lambda i: (i, 0)),
        pl.BlockSpec((K, D, D), lambda i: (0, 0, 0)),
        pl.BlockSpec((K, D), lambda i: (0, 0)),
    ],
    out_specs=[
        pl.BlockSpec((BLK, 1), lambda i: (i, 0)),
        pl.BlockSpec((BLK, D), lambda i: (i, 0)),
        pl.BlockSpec((K, BLK, D), lambda i: (0, i, 0)),
    ],
    out_shape=[
        jax.ShapeDtypeStruct((N, 1), jnp.float32),
        jax.ShapeDtypeStruct((N, D), jnp.float32),
        jax.ShapeDtypeStruct((K, N, D), jnp.float32),
    ],
)


def _mid_body(parts_ref, norm_ref, w0_ref, b0_ref, xv_ref, h1n_ref):
    norm = norm_ref[...]
    px = (parts_ref[0] + parts_ref[1]) * norm
    for k in range(K):
        h1 = (jnp.dot(px, w0_ref[k], preferred_element_type=jnp.float32)
              + b0_ref[k:k + 1, :] + xv_ref[k])
        h1n_ref[k] = h1 * norm


_mid_call = pl.pallas_call(
    _mid_body,
    grid=(_GRID,),
    in_specs=[
        pl.BlockSpec((NC, BLK, D), lambda i: (0, i, 0)),
        pl.BlockSpec((BLK, 1), lambda i: (i, 0)),
        pl.BlockSpec((K, D, D), lambda i: (0, 0, 0)),
        pl.BlockSpec((K, D), lambda i: (0, 0)),
        pl.BlockSpec((K, BLK, D), lambda i: (0, i, 0)),
    ],
    out_specs=pl.BlockSpec((K, BLK, D), lambda i: (0, i, 0)),
    out_shape=jax.ShapeDtypeStruct((K, N, D), jnp.float32),
)


def _final_body(agg_ref, norm_ref, w_ref, bw_ref, xv_ref, out_ref):
    norm = norm_ref[...]
    acc = jnp.zeros((BLK, D), jnp.float32)
    for k in range(K):
        t = agg_ref[k] * norm
        acc = acc + (jnp.dot(t, w_ref[k], preferred_element_type=jnp.float32)
                     + bw_ref[k:k + 1, :] + xv_ref[k])
    out_ref[...] = acc * (1.0 / K)


_final_call = pl.pallas_call(
    _final_body,
    grid=(_GRID,),
    in_specs=[
        pl.BlockSpec((K, BLK, D), lambda i: (0, i, 0)),
        pl.BlockSpec((BLK, 1), lambda i: (i, 0)),
        pl.BlockSpec((K, D, D), lambda i: (0, 0, 0)),
        pl.BlockSpec((K, D), lambda i: (0, 0)),
        pl.BlockSpec((K, BLK, D), lambda i: (0, i, 0)),
    ],
    out_specs=pl.BlockSpec((BLK, D), lambda i: (i, 0)),
    out_shape=jax.ShapeDtypeStruct((N, D), jnp.float32),
)


def kernel(x, edge_index, W0, b0, W, bW, V, bV):
    src = edge_index[0].astype(jnp.int32)
    dst = edge_index[1].astype(jnp.int32)
    pad = EP - E
    src = jnp.concatenate([src, jnp.zeros((pad,), jnp.int32)])
    dst = jnp.concatenate([dst, jnp.full((pad,), NP - 1, jnp.int32)])
    src3 = src.reshape(SNM, SSUBS, SUB)
    dst3 = dst.reshape(SNM, SSUBS, SUB)
    dst3d = dst.reshape(DNM, DSUBS, SUB)
    zeros128 = jnp.zeros((NP, D), jnp.float32)
    ones128 = jnp.ones((SUB, D), jnp.float32)

    degp = _deg_call(dst3d, zeros128, ones128)[:, :N, :]
    norm, xn, xv = _prep_call(degp, x, V, bV)

    src_split = jnp.stack([src3, src3])
    aggp = _spmm_split(xn, src_split, dst3, zeros128)[:, :N, :]
    h1n = _mid_call(aggp, norm, W0, b0, xv)

    src_perk = jnp.stack([src3, src3 + N])
    agg2 = _spmm_perk(h1n.reshape(K * N, D), src_perk, dst3, zeros128)[:, :N, :]
    return _final_call(agg2, norm, W, bW, xv)


# speedup vs baseline: 8.2770x; 2.9198x over previous
import functools

import jax
import jax.numpy as jnp
from jax import lax
from jax.experimental import pallas as pl
from jax.experimental.pallas import tpu as pltpu
from jax.experimental.pallas import tpu_sc as plsc

N = 10000
E = 320000
D = 128
K = 2

SUB = 128
EP = 327680
DSUBS = 4
DNM = EP // (SUB * DSUBS)
SSUBS = 2
SNM = EP // (SUB * SSUBS)
NC = 2
NS = 16
NW = NC * NS
NP = 10240
RPS = NP // NS

_MESH = plsc.VectorSubcoreMesh(core_axis_name="c", subcore_axis_name="s")


def _deg_body(dst_hbm, zeros_hbm, ones_hbm, out_hbm, dst_v, ones_v, acc_sh):
    c = lax.axis_index("c")
    s = lax.axis_index("s")
    w = c * NS + s
    pltpu.sync_copy(zeros_hbm.at[pl.ds(s * RPS, RPS)],
                    acc_sh.at[pl.ds(s * RPS, RPS)])
    pltpu.sync_copy(ones_hbm, ones_v)
    plsc.subcore_barrier()
    mpw = DNM // NW

    def body(i, _):
        m = w * mpw + i
        pltpu.sync_copy(dst_hbm.at[m], dst_v)
        for j in range(DSUBS):
            pltpu.sync_copy(ones_v, acc_sh.at[dst_v.at[j]], add=True)
        return ()

    lax.fori_loop(0, mpw, body, ())
    plsc.subcore_barrier()
    pltpu.sync_copy(acc_sh.at[pl.ds(s * RPS, RPS)],
                    out_hbm.at[c, pl.ds(s * RPS, RPS)])


_deg_call = pl.kernel(
    _deg_body,
    out_type=jax.ShapeDtypeStruct((NC, NP, D), jnp.float32),
    mesh=_MESH,
    scratch_types=[
        pltpu.VMEM((DSUBS, SUB), jnp.int32),
        pltpu.VMEM((SUB, D), jnp.float32),
        pltpu.VMEM_SHARED((NP, D), jnp.float32),
    ],
)


GM = 20
GT = GM * SSUBS


def _spmm_body(per_core, table_hbm, src_hbm, dst_hbm, zeros_hbm, out_hbm,
               src_v, dst_v, rows0_v, rows1_v, acc_sh, sem0, sem1):
    c = lax.axis_index("c")
    s = lax.axis_index("s")
    pltpu.sync_copy(zeros_hbm.at[pl.ds(s * RPS, RPS)],
                    acc_sh.at[pl.ds(s * RPS, RPS)])
    plsc.subcore_barrier()
    if per_core:
        mpw = SNM // NS
        base = s * mpw
    else:
        mpw = SNM // NW
        base = (c * NS + s) * mpw

    def group(g, _):
        t0 = (base + g * GM) * SSUBS
        pltpu.sync_copy(src_hbm.at[c, pl.ds(t0, GT)], src_v)
        pltpu.sync_copy(dst_hbm.at[pl.ds(t0, GT)], dst_v)
        pltpu.async_copy(table_hbm.at[src_v.at[0]], rows0_v, sem0)
        pltpu.async_copy(table_hbm.at[src_v.at[1]], rows1_v, sem1)

        def pair(i, _):
            t = 2 * i
            pltpu.make_async_copy(table_hbm.at[src_v.at[t]], rows0_v, sem0).wait()
            pltpu.sync_copy(rows0_v, acc_sh.at[dst_v.at[t]], add=True)

            @pl.when(t + 2 < GT)
            def _():
                pltpu.async_copy(table_hbm.at[src_v.at[t + 2]], rows0_v, sem0)

            pltpu.make_async_copy(table_hbm.at[src_v.at[t + 1]], rows1_v, sem1).wait()
            pltpu.sync_copy(rows1_v, acc_sh.at[dst_v.at[t + 1]], add=True)

            @pl.when(t + 3 < GT)
            def _():
                pltpu.async_copy(table_hbm.at[src_v.at[t + 3]], rows1_v, sem1)
            return ()

        lax.fori_loop(0, GT // 2, pair, ())
        return ()

    lax.fori_loop(0, mpw // GM, group, ())
    plsc.subcore_barrier()
    pltpu.sync_copy(acc_sh.at[pl.ds(s * RPS, RPS)],
                    out_hbm.at[c, pl.ds(s * RPS, RPS)])


def _make_spmm(per_core):
    return pl.kernel(
        functools.partial(_spmm_body, per_core),
        out_type=jax.ShapeDtypeStruct((NC, NP, D), jnp.float32),
        mesh=_MESH,
        scratch_types=[
            pltpu.VMEM((GT, SUB), jnp.int32),
            pltpu.VMEM((GT, SUB), jnp.int32),
            pltpu.VMEM((SUB, D), jnp.float32),
            pltpu.VMEM((SUB, D), jnp.float32),
            pltpu.VMEM_SHARED((NP, D), jnp.float32),
            pltpu.SemaphoreType.DMA,
            pltpu.SemaphoreType.DMA,
        ],
    )


_spmm_split = _make_spmm(False)
_spmm_perk = _make_spmm(True)


BLK = 1000
_GRID = N // BLK


def _prep_body(deg_ref, x_ref, v_ref, bv_ref, norm_ref, xn_ref, xv_ref):
    deg = deg_ref[0, :, 0:1] + deg_ref[1, :, 0:1]
    norm = lax.rsqrt(jnp.maximum(deg, 1.0))
    x = x_ref[...]
    norm_ref[...] = norm
    xn_ref[...] = x * norm
    for k in range(K):
        xv_ref[k] = (jnp.dot(x, v_ref[k], preferred_element_type=jnp.float32)
                     + bv_ref[k:k + 1, :])


_prep_call = pl.pallas_call(
    _prep_body,
    grid=(_GRID,),
    in_specs=[
        pl.BlockSpec((NC, BLK, D), lambda i: (0, i, 0)),
        pl.BlockSpec((BLK, D), lambda i: (i, 0)),
        pl.BlockSpec((K, D, D), lambda i: (0, 0, 0)),
        pl.BlockSpec((K, D), lambda i: (0, 0)),
    ],
    out_specs=[
        pl.BlockSpec((BLK, 1), lambda i: (i, 0)),
        pl.BlockSpec((BLK, D), lambda i: (i, 0)),
        pl.BlockSpec((K, BLK, D), lambda i: (0, i, 0)),
    ],
    out_shape=[
        jax.ShapeDtypeStruct((N, 1), jnp.float32),
        jax.ShapeDtypeStruct((N, D), jnp.float32),
        jax.ShapeDtypeStruct((K, N, D), jnp.float32),
    ],
)


def _mid_body(parts_ref, norm_ref, w0_ref, b0_ref, xv_ref, h1n_ref):
    norm = norm_ref[...]
    px = (parts_ref[0] + parts_ref[1]) * norm
    for k in range(K):
        h1 = (jnp.dot(px, w0_ref[k], preferred_element_type=jnp.float32)
              + b0_ref[k:k + 1, :] + xv_ref[k])
        h1n_ref[k] = h1 * norm


_mid_call = pl.pallas_call(
    _mid_body,
    grid=(_GRID,),
    in_specs=[
        pl.BlockSpec((NC, BLK, D), lambda i: (0, i, 0)),
        pl.BlockSpec((BLK, 1), lambda i: (i, 0)),
        pl.BlockSpec((K, D, D), lambda i: (0, 0, 0)),
        pl.BlockSpec((K, D), lambda i: (0, 0)),
        pl.BlockSpec((K, BLK, D), lambda i: (0, i, 0)),
    ],
    out_specs=pl.BlockSpec((K, BLK, D), lambda i: (0, i, 0)),
    out_shape=jax.ShapeDtypeStruct((K, N, D), jnp.float32),
)


def _final_body(agg_ref, norm_ref, w_ref, bw_ref, xv_ref, out_ref):
    norm = norm_ref[...]
    acc = jnp.zeros((BLK, D), jnp.float32)
    for k in range(K):
        t = agg_ref[k] * norm
        acc = acc + (jnp.dot(t, w_ref[k], preferred_element_type=jnp.float32)
                     + bw_ref[k:k + 1, :] + xv_ref[k])
    out_ref[...] = acc * (1.0 / K)


_final_call = pl.pallas_call(
    _final_body,
    grid=(_GRID,),
    in_specs=[
        pl.BlockSpec((K, BLK, D), lambda i: (0, i, 0)),
        pl.BlockSpec((BLK, 1), lambda i: (i, 0)),
        pl.BlockSpec((K, D, D), lambda i: (0, 0, 0)),
        pl.BlockSpec((K, D), lambda i: (0, 0)),
        pl.BlockSpec((K, BLK, D), lambda i: (0, i, 0)),
    ],
    out_specs=pl.BlockSpec((BLK, D), lambda i: (i, 0)),
    out_shape=jax.ShapeDtypeStruct((N, D), jnp.float32),
)


def kernel(x, edge_index, W0, b0, W, bW, V, bV):
    src = edge_index[0].astype(jnp.int32)
    dst = edge_index[1].astype(jnp.int32)
    pad = EP - E
    spread = jnp.arange(pad, dtype=jnp.int32) % (NP - N)
    src = jnp.concatenate([src, spread])
    dst = jnp.concatenate([dst, N + spread])
    src3 = src.reshape(SNM * SSUBS, SUB)
    dst3 = dst.reshape(SNM * SSUBS, SUB)
    dst3d = dst.reshape(DNM, DSUBS, SUB)
    zeros128 = jnp.zeros((NP, D), jnp.float32)
    ones128 = jnp.ones((SUB, D), jnp.float32)

    degp = _deg_call(dst3d, zeros128, ones128)[:, :N, :]
    norm, xn, xv = _prep_call(degp, x, V, bV)

    src_split = jnp.stack([src3, src3])
    aggp = _spmm_split(xn, src_split, dst3, zeros128)[:, :N, :]
    h1n = _mid_call(aggp, norm, W0, b0, xv)

    src_perk = jnp.stack([src3, src3 + N])
    agg2 = _spmm_perk(h1n.reshape(K * N, D), src_perk, dst3, zeros128)[:, :N, :]
    return _final_call(agg2, norm, W, bW, xv)


# speedup vs baseline: 8.6642x; 1.0468x over previous
import functools

import jax
import jax.numpy as jnp
from jax import lax
from jax.experimental import pallas as pl
from jax.experimental.pallas import tpu as pltpu
from jax.experimental.pallas import tpu_sc as plsc

N = 10000
E = 320000
D = 128
K = 2

SUB = 128
EP = 327680
DSUBS = 4
DNM = EP // (SUB * DSUBS)
SSUBS = 2
SNM = EP // (SUB * SSUBS)
NC = 2
NS = 16
NW = NC * NS
NP = 10240
RPS = NP // NS

_MESH = plsc.VectorSubcoreMesh(core_axis_name="c", subcore_axis_name="s")


DT = (DNM // NW) * DSUBS


def _deg_body(dst_hbm, zeros_hbm, ones_hbm, out_hbm, dst_v, ones_v, acc_sh):
    c = lax.axis_index("c")
    s = lax.axis_index("s")
    w = c * NS + s
    pltpu.sync_copy(zeros_hbm.at[pl.ds(s * RPS, RPS)],
                    acc_sh.at[pl.ds(s * RPS, RPS)])
    pltpu.sync_copy(ones_hbm, ones_v)
    pltpu.sync_copy(dst_hbm.at[c, pl.ds(s * DT, DT)], dst_v)
    plsc.subcore_barrier()

    def body(i, _):
        pltpu.sync_copy(ones_v, acc_sh.at[dst_v.at[i]], add=True)
        return ()

    lax.fori_loop(0, DT, body, ())
    plsc.subcore_barrier()
    pltpu.sync_copy(acc_sh.at[pl.ds(s * RPS, RPS)],
                    out_hbm.at[c, pl.ds(s * RPS, RPS)])


_deg_call = pl.kernel(
    _deg_body,
    out_type=jax.ShapeDtypeStruct((NC, NP, D), jnp.float32),
    mesh=_MESH,
    scratch_types=[
        pltpu.VMEM((DT, SUB), jnp.int32),
        pltpu.VMEM((SUB, D), jnp.float32),
        pltpu.VMEM_SHARED((NP, D), jnp.float32),
    ],
)


GM = 20
GT = GM * SSUBS


def _spmm_body(per_core, table_hbm, src_hbm, dst_hbm, zeros_hbm, out_hbm,
               src_v, dst_v, rows0_v, rows1_v, acc_sh, sem0, sem1):
    c = lax.axis_index("c")
    s = lax.axis_index("s")
    pltpu.sync_copy(zeros_hbm.at[pl.ds(s * RPS, RPS)],
                    acc_sh.at[pl.ds(s * RPS, RPS)])
    plsc.subcore_barrier()
    if per_core:
        mpw = SNM // NS
        base = s * mpw
    else:
        mpw = SNM // NW
        base = (c * NS + s) * mpw

    def group(g, _):
        t0 = (base + g * GM) * SSUBS
        pltpu.sync_copy(src_hbm.at[c, pl.ds(t0, GT)], src_v)
        pltpu.sync_copy(dst_hbm.at[pl.ds(t0, GT)], dst_v)
        pltpu.async_copy(table_hbm.at[src_v.at[0]], rows0_v, sem0)
        pltpu.async_copy(table_hbm.at[src_v.at[1]], rows1_v, sem1)

        def pair(i, _):
            t = 2 * i
            pltpu.make_async_copy(table_hbm.at[src_v.at[t]], rows0_v, sem0).wait()
            pltpu.sync_copy(rows0_v, acc_sh.at[dst_v.at[t]], add=True)

            @pl.when(t + 2 < GT)
            def _():
                pltpu.async_copy(table_hbm.at[src_v.at[t + 2]], rows0_v, sem0)

            pltpu.make_async_copy(table_hbm.at[src_v.at[t + 1]], rows1_v, sem1).wait()
            pltpu.sync_copy(rows1_v, acc_sh.at[dst_v.at[t + 1]], add=True)

            @pl.when(t + 3 < GT)
            def _():
                pltpu.async_copy(table_hbm.at[src_v.at[t + 3]], rows1_v, sem1)
            return ()

        lax.fori_loop(0, GT // 2, pair, ())
        return ()

    lax.fori_loop(0, mpw // GM, group, ())
    plsc.subcore_barrier()
    pltpu.sync_copy(acc_sh.at[pl.ds(s * RPS, RPS)],
                    out_hbm.at[c, pl.ds(s * RPS, RPS)])


def _make_spmm(per_core):
    return pl.kernel(
        functools.partial(_spmm_body, per_core),
        out_type=jax.ShapeDtypeStruct((NC, NP, D), jnp.float32),
        mesh=_MESH,
        scratch_types=[
            pltpu.VMEM((GT, SUB), jnp.int32),
            pltpu.VMEM((GT, SUB), jnp.int32),
            pltpu.VMEM((SUB, D), jnp.float32),
            pltpu.VMEM((SUB, D), jnp.float32),
            pltpu.VMEM_SHARED((NP, D), jnp.float32),
            pltpu.SemaphoreType.DMA,
            pltpu.SemaphoreType.DMA,
        ],
    )


_spmm_split = _make_spmm(False)
_spmm_perk = _make_spmm(True)


BLK = 1024
_GRID = NP // BLK


def _prep_body(deg_ref, x_ref, v_ref, bv_ref, norm_ref, xn_ref, xv_ref):
    deg = deg_ref[0, :, 0:1] + deg_ref[1, :, 0:1]
    norm = lax.rsqrt(jnp.maximum(deg, 1.0))
    x = x_ref[...]
    norm_ref[...] = norm
    xn_ref[...] = x * norm
    for k in range(K):
        xv_ref[k] = (jnp.dot(x, v_ref[k], preferred_element_type=jnp.float32)
                     + bv_ref[k:k + 1, :])


_prep_call = pl.pallas_call(
    _prep_body,
    grid=(_GRID,),
    in_specs=[
        pl.BlockSpec((NC, BLK, D), lambda i: (0, i, 0)),
        pl.BlockSpec((BLK, D), lambda i: (i, 0)),
        pl.BlockSpec((K, D, D), lambda i: (0, 0, 0)),
        pl.BlockSpec((K, D), lambda i: (0, 0)),
    ],
    out_specs=[
        pl.BlockSpec((BLK, 1), lambda i: (i, 0)),
        pl.BlockSpec((BLK, D), lambda i: (i, 0)),
        pl.BlockSpec((K, BLK, D), lambda i: (0, i, 0)),
    ],
    out_shape=[
        jax.ShapeDtypeStruct((NP, 1), jnp.float32),
        jax.ShapeDtypeStruct((NP, D), jnp.float32),
        jax.ShapeDtypeStruct((K, NP, D), jnp.float32),
    ],
)


def _mid_body(parts_ref, norm_ref, w0_ref, b0_ref, xv_ref, h1n_ref):
    norm = norm_ref[...]
    px = (parts_ref[0] + parts_ref[1]) * norm
    for k in range(K):
        h1 = (jnp.dot(px, w0_ref[k], preferred_element_type=jnp.float32)
              + b0_ref[k:k + 1, :] + xv_ref[k])
        h1n_ref[k] = h1 * norm


_mid_call = pl.pallas_call(
    _mid_body,
    grid=(_GRID,),
    in_specs=[
        pl.BlockSpec((NC, BLK, D), lambda i: (0, i, 0)),
        pl.BlockSpec((BLK, 1), lambda i: (i, 0)),
        pl.BlockSpec((K, D, D), lambda i: (0, 0, 0)),
        pl.BlockSpec((K, D), lambda i: (0, 0)),
        pl.BlockSpec((K, BLK, D), lambda i: (0, i, 0)),
    ],
    out_specs=pl.BlockSpec((K, BLK, D), lambda i: (0, i, 0)),
    out_shape=jax.ShapeDtypeStruct((K, NP, D), jnp.float32),
)


def _final_body(agg_ref, norm_ref, w_ref, bw_ref, xv_ref, out_ref):
    norm = norm_ref[...]
    acc = jnp.zeros((BLK, D), jnp.float32)
    for k in range(K):
        t = agg_ref[k] * norm
        acc = acc + (jnp.dot(t, w_ref[k], preferred_element_type=jnp.float32)
                     + bw_ref[k:k + 1, :] + xv_ref[k])
    out_ref[...] = acc * (1.0 / K)


_final_call = pl.pallas_call(
    _final_body,
    grid=(_GRID,),
    in_specs=[
        pl.BlockSpec((K, BLK, D), lambda i: (0, i, 0)),
        pl.BlockSpec((BLK, 1), lambda i: (i, 0)),
        pl.BlockSpec((K, D, D), lambda i: (0, 0, 0)),
        pl.BlockSpec((K, D), lambda i: (0, 0)),
        pl.BlockSpec((K, BLK, D), lambda i: (0, i, 0)),
    ],
    out_specs=pl.BlockSpec((BLK, D), lambda i: (i, 0)),
    out_shape=jax.ShapeDtypeStruct((NP, D), jnp.float32),
)


def kernel(x, edge_index, W0, b0, W, bW, V, bV):
    src = edge_index[0].astype(jnp.int32)
    dst = edge_index[1].astype(jnp.int32)
    pad = EP - E
    spread = jnp.arange(pad, dtype=jnp.int32) % (NP - N)
    src = jnp.concatenate([src, spread])
    dst = jnp.concatenate([dst, N + spread])
    src3 = src.reshape(SNM * SSUBS, SUB)
    dst3 = dst.reshape(SNM * SSUBS, SUB)
    dst3d = dst.reshape(NC, (DNM // NC) * DSUBS, SUB)
    zeros128 = jnp.zeros((NP, D), jnp.float32)
    ones128 = jnp.ones((SUB, D), jnp.float32)

    x_pad = jnp.concatenate([x, jnp.zeros((NP - N, D), jnp.float32)])
    degp = _deg_call(dst3d, zeros128, ones128)
    norm, xn, xv = _prep_call(degp, x_pad, V, bV)

    src_split = jnp.stack([src3, src3])
    aggp = _spmm_split(xn, src_split, dst3, zeros128)
    h1n = _mid_call(aggp, norm, W0, b0, xv)

    src_perk = jnp.stack([src3, src3 + NP])
    agg2 = _spmm_perk(h1n.reshape(K * NP, D), src_perk, dst3, zeros128)
    return _final_call(agg2, norm, W, bW, xv)[:N]


# speedup vs baseline: 8.7791x; 1.0133x over previous
import functools

import jax
import jax.numpy as jnp
from jax import lax
from jax.experimental import pallas as pl
from jax.experimental.pallas import tpu as pltpu
from jax.experimental.pallas import tpu_sc as plsc

N = 10000
E = 320000
D = 128
K = 2

SUB = 128
EP = 327680
DSUBS = 4
DNM = EP // (SUB * DSUBS)
SSUBS = 2
SNM = EP // (SUB * SSUBS)
NC = 2
NS = 16
NW = NC * NS
NP = 10240
RPS = NP // NS

_MESH = plsc.VectorSubcoreMesh(core_axis_name="c", subcore_axis_name="s")


DT = (DNM // NW) * DSUBS


def _deg_body(dst_hbm, zeros_hbm, ones_hbm, out_hbm, dst_v, ones_v, acc_sh):
    c = lax.axis_index("c")
    s = lax.axis_index("s")
    w = c * NS + s
    pltpu.sync_copy(zeros_hbm.at[pl.ds(s * RPS, RPS)],
                    acc_sh.at[pl.ds(s * RPS, RPS)])
    pltpu.sync_copy(ones_hbm, ones_v)
    pltpu.sync_copy(dst_hbm.at[c, pl.ds(s * DT, DT)], dst_v)
    plsc.subcore_barrier()

    def body(i, _):
        pltpu.sync_copy(ones_v, acc_sh.at[dst_v.at[i]], add=True)
        return ()

    lax.fori_loop(0, DT, body, ())
    plsc.subcore_barrier()
    pltpu.sync_copy(acc_sh.at[pl.ds(s * RPS, RPS)],
                    out_hbm.at[c, pl.ds(s * RPS, RPS)])


_deg_call = pl.kernel(
    _deg_body,
    out_type=jax.ShapeDtypeStruct((NC, NP, D), jnp.float32),
    mesh=_MESH,
    scratch_types=[
        pltpu.VMEM((DT, SUB), jnp.int32),
        pltpu.VMEM((SUB, D), jnp.float32),
        pltpu.VMEM_SHARED((NP, D), jnp.float32),
    ],
)


GM = 20
GT = GM * SSUBS


def _spmm_body(per_core, table_hbm, src_hbm, dst_hbm, zeros_hbm, out_hbm,
               src_v, dst_v, rows0_v, rows1_v, acc_sh, sem0, sem1):
    c = lax.axis_index("c")
    s = lax.axis_index("s")
    pltpu.sync_copy(zeros_hbm.at[pl.ds(s * RPS, RPS)],
                    acc_sh.at[pl.ds(s * RPS, RPS)])
    plsc.subcore_barrier()
    if per_core:
        mpw = SNM // NS
        base = s * mpw
    else:
        mpw = SNM // NW
        base = (c * NS + s) * mpw

    def group(g, _):
        t0 = (base + g * GM) * SSUBS
        pltpu.sync_copy(src_hbm.at[c, pl.ds(t0, GT)], src_v)
        pltpu.sync_copy(dst_hbm.at[pl.ds(t0, GT)], dst_v)
        pltpu.async_copy(table_hbm.at[src_v.at[0]], rows0_v, sem0)
        pltpu.async_copy(table_hbm.at[src_v.at[1]], rows1_v, sem1)

        def pair(i, _):
            t = 2 * i
            pltpu.make_async_copy(table_hbm.at[src_v.at[t]], rows0_v, sem0).wait()
            pltpu.sync_copy(rows0_v, acc_sh.at[dst_v.at[t]], add=True)

            @pl.when(t + 2 < GT)
            def _():
                pltpu.async_copy(table_hbm.at[src_v.at[t + 2]], rows0_v, sem0)

            pltpu.make_async_copy(table_hbm.at[src_v.at[t + 1]], rows1_v, sem1).wait()
            pltpu.sync_copy(rows1_v, acc_sh.at[dst_v.at[t + 1]], add=True)

            @pl.when(t + 3 < GT)
            def _():
                pltpu.async_copy(table_hbm.at[src_v.at[t + 3]], rows1_v, sem1)
            return ()

        lax.fori_loop(0, GT // 2, pair, ())
        return ()

    lax.fori_loop(0, mpw // GM, group, ())
    plsc.subcore_barrier()
    pltpu.sync_copy(acc_sh.at[pl.ds(s * RPS, RPS)],
                    out_hbm.at[c, pl.ds(s * RPS, RPS)])


def _make_spmm(per_core):
    return pl.kernel(
        functools.partial(_spmm_body, per_core),
        out_type=jax.ShapeDtypeStruct((NC, NP, D), jnp.float32),
        mesh=_MESH,
        scratch_types=[
            pltpu.VMEM((GT, SUB), jnp.int32),
            pltpu.VMEM((GT, SUB), jnp.int32),
            pltpu.VMEM((SUB, D), jnp.float32),
            pltpu.VMEM((SUB, D), jnp.float32),
            pltpu.VMEM_SHARED((NP, D), jnp.float32),
            pltpu.SemaphoreType.DMA,
            pltpu.SemaphoreType.DMA,
        ],
    )


_spmm_split = _make_spmm(False)
_spmm_perk = _make_spmm(True)


BLK = 1024
_GRID = NP // BLK


def _prep_body(deg_ref, x_ref, norm_ref, xn_ref):
    deg = deg_ref[0, :, 0:1] + deg_ref[1, :, 0:1]
    norm = lax.rsqrt(jnp.maximum(deg, 1.0))
    norm_ref[...] = norm
    xn_ref[...] = x_ref[...] * norm


_prep_call = pl.pallas_call(
    _prep_body,
    grid=(_GRID,),
    in_specs=[
        pl.BlockSpec((NC, BLK, D), lambda i: (0, i, 0)),
        pl.BlockSpec((BLK, D), lambda i: (i, 0)),
    ],
    out_specs=[
        pl.BlockSpec((BLK, 1), lambda i: (i, 0)),
        pl.BlockSpec((BLK, D), lambda i: (i, 0)),
    ],
    out_shape=[
        jax.ShapeDtypeStruct((NP, 1), jnp.float32),
        jax.ShapeDtypeStruct((NP, D), jnp.float32),
    ],
)


def _mid_body(parts_ref, norm_ref, w0_ref, b0_ref, v_ref, bv_ref, x_ref,
              h1n_ref):
    norm = norm_ref[...]
    x = x_ref[...]
    px = (parts_ref[0] + parts_ref[1]) * norm
    for k in range(K):
        xv = (jnp.dot(x, v_ref[k], preferred_element_type=jnp.float32)
              + bv_ref[k:k + 1, :])
        h1 = (jnp.dot(px, w0_ref[k], preferred_element_type=jnp.float32)
              + b0_ref[k:k + 1, :] + xv)
        h1n_ref[k] = h1 * norm


_mid_call = pl.pallas_call(
    _mid_body,
    grid=(_GRID,),
    in_specs=[
        pl.BlockSpec((NC, BLK, D), lambda i: (0, i, 0)),
        pl.BlockSpec((BLK, 1), lambda i: (i, 0)),
        pl.BlockSpec((K, D, D), lambda i: (0, 0, 0)),
        pl.BlockSpec((K, D), lambda i: (0, 0)),
        pl.BlockSpec((K, D, D), lambda i: (0, 0, 0)),
        pl.BlockSpec((K, D), lambda i: (0, 0)),
        pl.BlockSpec((BLK, D), lambda i: (i, 0)),
    ],
    out_specs=pl.BlockSpec((K, BLK, D), lambda i: (0, i, 0)),
    out_shape=jax.ShapeDtypeStruct((K, NP, D), jnp.float32),
)


def _final_body(agg_ref, norm_ref, w_ref, bw_ref, v_ref, bv_ref, x_ref,
                out_ref):
    norm = norm_ref[...]
    x = x_ref[...]
    acc = jnp.zeros((BLK, D), jnp.float32)
    for k in range(K):
        xv = (jnp.dot(x, v_ref[k], preferred_element_type=jnp.float32)
              + bv_ref[k:k + 1, :])
        t = agg_ref[k] * norm
        acc = acc + (jnp.dot(t, w_ref[k], preferred_element_type=jnp.float32)
                     + bw_ref[k:k + 1, :] + xv)
    out_ref[...] = acc * (1.0 / K)


_final_call = pl.pallas_call(
    _final_body,
    grid=(_GRID,),
    in_specs=[
        pl.BlockSpec((K, BLK, D), lambda i: (0, i, 0)),
        pl.BlockSpec((BLK, 1), lambda i: (i, 0)),
        pl.BlockSpec((K, D, D), lambda i: (0, 0, 0)),
        pl.BlockSpec((K, D), lambda i: (0, 0)),
        pl.BlockSpec((K, D, D), lambda i: (0, 0, 0)),
        pl.BlockSpec((K, D), lambda i: (0, 0)),
        pl.BlockSpec((BLK, D), lambda i: (i, 0)),
    ],
    out_specs=pl.BlockSpec((BLK, D), lambda i: (i, 0)),
    out_shape=jax.ShapeDtypeStruct((NP, D), jnp.float32),
)


def kernel(x, edge_index, W0, b0, W, bW, V, bV):
    src = edge_index[0].astype(jnp.int32)
    dst = edge_index[1].astype(jnp.int32)
    pad = EP - E
    spread = jnp.arange(pad, dtype=jnp.int32) % (NP - N)
    src = jnp.concatenate([src, spread])
    dst = jnp.concatenate([dst, N + spread])
    src3 = src.reshape(SNM * SSUBS, SUB)
    dst3 = dst.reshape(SNM * SSUBS, SUB)
    dst3d = dst.reshape(NC, (DNM // NC) * DSUBS, SUB)
    zeros128 = jnp.zeros((NP, D), jnp.float32)
    ones128 = jnp.ones((SUB, D), jnp.float32)

    x_pad = jnp.concatenate([x, jnp.zeros((NP - N, D), jnp.float32)])
    degp = _deg_call(dst3d, zeros128, ones128)
    norm, xn = _prep_call(degp, x_pad)

    src_split = jnp.stack([src3, src3])
    aggp = _spmm_split(xn, src_split, dst3, zeros128)
    h1n = _mid_call(aggp, norm, W0, b0, V, bV, x_pad)

    src_perk = jnp.stack([src3, src3 + NP])
    agg2 = _spmm_perk(h1n.reshape(K * NP, D), src_perk, dst3, zeros128)
    return _final_call(agg2, norm, W, bW, V, bV, x_pad)[:N]
